# baseline (speedup 1.0000x reference)
import functools

import jax
import jax.numpy as jnp
from jax import lax
from jax.experimental import pallas as pl
from jax.experimental.pallas import tpu as pltpu
from jax.experimental.pallas import tpu_sc as plsc

BATCH = 4096
SEQ = 200
SEQ_PAD = 256
D = 64
DP = 128
VOCAB = 1000000
NC = 2
NS = 16
NW = NC * NS
CHUNKS = BATCH // NW
NBUF = 4
STEPS = CHUNKS // NBUF
GATHER_SLICES = ((0, 128), (128, 72))
ROW_UNROLL = 8
TBLK = 4096
FMT_GRID = -(-VOCAB // TBLK)


def _fmt_body(tt_ref, out_ref):
    eye = jnp.eye(D, dtype=jnp.float32)
    out_ref[:, 0:D] = jax.lax.dot_general(
        tt_ref[...], eye, (((0,), (0,)), ((), ())),
        preferred_element_type=jnp.float32)


_fmt_kernel = pl.pallas_call(
    _fmt_body,
    grid=(FMT_GRID,),
    in_specs=[pl.BlockSpec((D, TBLK), lambda j: (0, j))],
    out_specs=pl.BlockSpec((TBLK, DP), lambda j: (j, 0)),
    out_shape=jax.ShapeDtypeStruct((VOCAB, DP), jnp.float32),
)


def _emb_body(x_hbm, tok_hbm, pos_hbm, out_hbm, idx_v, buf, pos_v, sems):
    cid = lax.axis_index("c")
    sid = lax.axis_index("s")
    wid = sid * NC + cid
    base_b = wid * CHUNKS
    sem_i, sem_g, sem_o = sems

    pltpu.sync_copy(pos_hbm, pos_v)

    def issue_idx(c, k):
        pltpu.async_copy(x_hbm.at[base_b + c], idx_v.at[k], sem_i.at[k])

    def wait_idx(k):
        pltpu.make_async_copy(x_hbm.at[0], idx_v.at[k], sem_i.at[k]).wait()

    def issue_gathers(k):
        for (o, n) in GATHER_SLICES:
            pltpu.async_copy(
                tok_hbm.at[idx_v.at[k, pl.ds(o, n)]],
                buf.at[k, pl.ds(o, n)],
                sem_g.at[k],
            )

    def wait_gathers(k):
        pltpu.make_async_copy(tok_hbm.at[pl.ds(0, SEQ)], buf.at[k], sem_g.at[k]).wait()

    def issue_out(c, k):
        pltpu.async_copy(buf.at[k], out_hbm.at[base_b + c], sem_o.at[k])

    def wait_out(k):
        pltpu.make_async_copy(buf.at[k], out_hbm.at[0], sem_o.at[k]).wait()

    def add_pos(k):
        def body(i, carry):
            r0 = i * ROW_UNROLL
            for rr in range(ROW_UNROLL):
                for cc in range(D // 16):
                    plsc.addupdate(
                        buf.at[k, r0 + rr, pl.ds(cc * 16, 16)],
                        pos_v[r0 + rr, pl.ds(cc * 16, 16)],
                    )
            return carry
        lax.fori_loop(0, SEQ // ROW_UNROLL, body, 0)

    for k in range(NBUF):
        issue_idx(k, k)
    for k in range(NBUF):
        wait_idx(k)
        issue_gathers(k)
    for k in range(NBUF):
        wait_gathers(k)
        issue_idx(k + NBUF, k)
        add_pos(k)
        issue_out(k, k)

    def turn(t, carry):
        for k in range(NBUF):
            wait_idx(k)
            wait_out(k)
            issue_gathers(k)
        for k in range(NBUF):
            c = t * NBUF + k
            wait_gathers(k)
            issue_idx(jnp.minimum(c + NBUF, CHUNKS - 1), k)
            add_pos(k)
            issue_out(c, k)
        return carry

    lax.fori_loop(1, STEPS, turn, 0)

    for k in range(NBUF):
        wait_out(k)
        wait_idx(k)


@functools.partial(
    pl.kernel,
    out_type=jax.ShapeDtypeStruct((BATCH, SEQ, DP), jnp.float32),
    mesh=plsc.VectorSubcoreMesh(core_axis_name="c", subcore_axis_name="s"),
    compiler_params=pltpu.CompilerParams(use_tc_tiling_on_sc=True),
    scratch_types=[
        pltpu.VMEM((NBUF, SEQ_PAD), jnp.int32),
        pltpu.VMEM((NBUF, SEQ, DP), jnp.float32),
        pltpu.VMEM((SEQ, DP), jnp.float32),
        (
            pltpu.SemaphoreType.DMA((NBUF,)),
            pltpu.SemaphoreType.DMA((NBUF,)),
            pltpu.SemaphoreType.DMA((NBUF,)),
        ),
    ],
)
def _emb_kernel(x_hbm, tok_hbm, pos_hbm, out_hbm, idx_v, buf, pos_v, sems):
    _emb_body(x_hbm, tok_hbm, pos_hbm, out_hbm, idx_v, buf, pos_v, sems)


def kernel(x, token_table, pos_table):
    tfmt = _fmt_kernel(token_table.T)
    xp = jnp.pad(x, ((0, 0), (0, SEQ_PAD - SEQ)))
    pp = jnp.pad(pos_table, ((0, 0), (0, DP - D)))
    out = _emb_kernel(xp, tfmt, pp)
    return out[:, :, :D]


# speedup vs baseline: 1.0118x; 1.0118x over previous
import functools

import jax
import jax.numpy as jnp
from jax import lax
from jax.experimental import pallas as pl
from jax.experimental.pallas import tpu as pltpu
from jax.experimental.pallas import tpu_sc as plsc

BATCH = 4096
SEQ = 200
SEQ_PAD = 256
D = 64
DP = 128
VOCAB = 1000000
NC = 2
NS = 16
NW = NC * NS
CHUNKS = BATCH // NW
NBUF = 4
STEPS = CHUNKS // NBUF
GATHER_SLICES = ((0, 128), (128, 72))
ROW_UNROLL = 8
TBLK = 4096
FMT_GRID = -(-VOCAB // TBLK)


def _fmt_body(tt_ref, out_ref):
    out_ref[:, 0:D] = tt_ref[...].T


_fmt_kernel = pl.pallas_call(
    _fmt_body,
    grid=(FMT_GRID,),
    in_specs=[pl.BlockSpec((D, TBLK), lambda j: (0, j))],
    out_specs=pl.BlockSpec((TBLK, DP), lambda j: (j, 0)),
    out_shape=jax.ShapeDtypeStruct((VOCAB, DP), jnp.float32),
)


def _emb_body(x_hbm, tok_hbm, pos_hbm, out_hbm, idx_v, buf, pos_v, sems):
    cid = lax.axis_index("c")
    sid = lax.axis_index("s")
    wid = sid * NC + cid
    base_b = wid * CHUNKS
    sem_i, sem_g, sem_o = sems

    pltpu.sync_copy(pos_hbm, pos_v)

    def issue_idx(c, k):
        pltpu.async_copy(x_hbm.at[base_b + c], idx_v.at[k], sem_i.at[k])

    def wait_idx(k):
        pltpu.make_async_copy(x_hbm.at[0], idx_v.at[k], sem_i.at[k]).wait()

    def issue_gathers(k):
        for (o, n) in GATHER_SLICES:
            pltpu.async_copy(
                tok_hbm.at[idx_v.at[k, pl.ds(o, n)]],
                buf.at[k, pl.ds(o, n)],
                sem_g.at[k],
            )

    def wait_gathers(k):
        pltpu.make_async_copy(tok_hbm.at[pl.ds(0, SEQ)], buf.at[k], sem_g.at[k]).wait()

    def issue_out(c, k):
        pltpu.async_copy(buf.at[k], out_hbm.at[base_b + c], sem_o.at[k])

    def wait_out(k):
        pltpu.make_async_copy(buf.at[k], out_hbm.at[0], sem_o.at[k]).wait()

    def add_pos(k):
        def body(i, carry):
            r0 = i * ROW_UNROLL
            for rr in range(ROW_UNROLL):
                for cc in range(D // 16):
                    plsc.addupdate(
                        buf.at[k, r0 + rr, pl.ds(cc * 16, 16)],
                        pos_v[r0 + rr, pl.ds(cc * 16, 16)],
                    )
            return carry
        lax.fori_loop(0, SEQ // ROW_UNROLL, body, 0)

    for k in range(NBUF):
        issue_idx(k, k)
    for k in range(NBUF):
        wait_idx(k)
        issue_gathers(k)
    for k in range(NBUF):
        wait_gathers(k)
        issue_idx(k + NBUF, k)
        add_pos(k)
        issue_out(k, k)

    def turn(t, carry):
        for k in range(NBUF):
            wait_idx(k)
            wait_out(k)
            issue_gathers(k)
        for k in range(NBUF):
            c = t * NBUF + k
            wait_gathers(k)
            issue_idx(jnp.minimum(c + NBUF, CHUNKS - 1), k)
            add_pos(k)
            issue_out(c, k)
        return carry

    lax.fori_loop(1, STEPS, turn, 0)

    for k in range(NBUF):
        wait_out(k)
        wait_idx(k)


@functools.partial(
    pl.kernel,
    out_type=jax.ShapeDtypeStruct((BATCH, SEQ, DP), jnp.float32),
    mesh=plsc.VectorSubcoreMesh(core_axis_name="c", subcore_axis_name="s"),
    compiler_params=pltpu.CompilerParams(use_tc_tiling_on_sc=True),
    scratch_types=[
        pltpu.VMEM((NBUF, SEQ_PAD), jnp.int32),
        pltpu.VMEM((NBUF, SEQ, DP), jnp.float32),
        pltpu.VMEM((SEQ, DP), jnp.float32),
        (
            pltpu.SemaphoreType.DMA((NBUF,)),
            pltpu.SemaphoreType.DMA((NBUF,)),
            pltpu.SemaphoreType.DMA((NBUF,)),
        ),
    ],
)
def _emb_kernel(x_hbm, tok_hbm, pos_hbm, out_hbm, idx_v, buf, pos_v, sems):
    _emb_body(x_hbm, tok_hbm, pos_hbm, out_hbm, idx_v, buf, pos_v, sems)


def kernel(x, token_table, pos_table):
    tfmt = _fmt_kernel(token_table.T)
    xp = jnp.pad(x, ((0, 0), (0, SEQ_PAD - SEQ)))
    pp = jnp.pad(pos_table, ((0, 0), (0, DP - D)))
    out = _emb_kernel(xp, tfmt, pp)
    return out[:, :, :D]


# speedup vs baseline: 1.1351x; 1.1219x over previous
import functools

import jax
import jax.numpy as jnp
from jax import lax
from jax.experimental import pallas as pl
from jax.experimental.pallas import tpu as pltpu
from jax.experimental.pallas import tpu_sc as plsc

BATCH = 4096
SEQ = 200
SEQ_PAD = 256
D = 64
DP = 128
VOCAB = 1000000
NC = 2
NS = 16
NW = NC * NS
CHUNKS = BATCH // NW
NBUF = 4
STEPS = CHUNKS // NBUF
GATHER_SLICES = ((0, 128), (128, 72))
ROW_UNROLL = 8
TBLK = 16384
FMT_GRID = -(-VOCAB // TBLK)


def _fmt_body(tt_ref, out_ref):
    out_ref[:, 0:D] = tt_ref[...].T


_fmt_kernel = pl.pallas_call(
    _fmt_body,
    grid=(FMT_GRID,),
    in_specs=[pl.BlockSpec((D, TBLK), lambda j: (0, j))],
    out_specs=pl.BlockSpec((TBLK, DP), lambda j: (j, 0)),
    out_shape=jax.ShapeDtypeStruct((VOCAB, DP), jnp.float32),
)


def _emb_body(x_hbm, tok_hbm, pos_hbm, out_hbm, idx_v, buf, pos_v, sems):
    cid = lax.axis_index("c")
    sid = lax.axis_index("s")
    wid = sid * NC + cid
    base_b = wid * CHUNKS
    sem_i, sem_g, sem_o = sems

    pltpu.sync_copy(pos_hbm, pos_v)

    def issue_idx(c, k):
        pltpu.async_copy(x_hbm.at[base_b + c], idx_v.at[k], sem_i.at[k])

    def wait_idx(k):
        pltpu.make_async_copy(x_hbm.at[0], idx_v.at[k], sem_i.at[k]).wait()

    def issue_gathers(k):
        for (o, n) in GATHER_SLICES:
            pltpu.async_copy(
                tok_hbm.at[idx_v.at[k, pl.ds(o, n)]],
                buf.at[k, pl.ds(o, n)],
                sem_g.at[k],
            )

    def wait_gathers(k):
        pltpu.make_async_copy(tok_hbm.at[pl.ds(0, SEQ)], buf.at[k], sem_g.at[k]).wait()

    def issue_out(c, k):
        pltpu.async_copy(buf.at[k], out_hbm.at[base_b + c], sem_o.at[k])

    def wait_out(k):
        pltpu.make_async_copy(buf.at[k], out_hbm.at[0], sem_o.at[k]).wait()

    def add_pos(k):
        def body(i, carry):
            r0 = i * ROW_UNROLL
            for rr in range(ROW_UNROLL):
                for cc in range(D // 16):
                    plsc.addupdate(
                        buf.at[k, r0 + rr, pl.ds(cc * 16, 16)],
                        pos_v[r0 + rr, pl.ds(cc * 16, 16)],
                    )
            return carry
        lax.fori_loop(0, SEQ // ROW_UNROLL, body, 0)

    for k in range(NBUF):
        issue_idx(k, k)
    for k in range(NBUF):
        wait_idx(k)
        issue_gathers(k)
    for k in range(NBUF):
        wait_gathers(k)
        issue_idx(k + NBUF, k)
        add_pos(k)
        issue_out(k, k)

    def turn(t, carry):
        for k in range(NBUF):
            wait_idx(k)
            wait_out(k)
            issue_gathers(k)
        for k in range(NBUF):
            c = t * NBUF + k
            wait_gathers(k)
            issue_idx(jnp.minimum(c + NBUF, CHUNKS - 1), k)
            add_pos(k)
            issue_out(c, k)
        return carry

    lax.fori_loop(1, STEPS, turn, 0)

    for k in range(NBUF):
        wait_out(k)
        wait_idx(k)


@functools.partial(
    pl.kernel,
    out_type=jax.ShapeDtypeStruct((BATCH, SEQ, DP), jnp.float32),
    mesh=plsc.VectorSubcoreMesh(core_axis_name="c", subcore_axis_name="s"),
    compiler_params=pltpu.CompilerParams(use_tc_tiling_on_sc=True),
    scratch_types=[
        pltpu.VMEM((NBUF, SEQ_PAD), jnp.int32),
        pltpu.VMEM((NBUF, SEQ, DP), jnp.float32),
        pltpu.VMEM((SEQ, DP), jnp.float32),
        (
            pltpu.SemaphoreType.DMA((NBUF,)),
            pltpu.SemaphoreType.DMA((NBUF,)),
            pltpu.SemaphoreType.DMA((NBUF,)),
        ),
    ],
)
def _emb_kernel(x_hbm, tok_hbm, pos_hbm, out_hbm, idx_v, buf, pos_v, sems):
    _emb_body(x_hbm, tok_hbm, pos_hbm, out_hbm, idx_v, buf, pos_v, sems)


def kernel(x, token_table, pos_table):
    tfmt = _fmt_kernel(token_table.T)
    xp = jnp.pad(x, ((0, 0), (0, SEQ_PAD - SEQ)))
    pp = jnp.pad(pos_table, ((0, 0), (0, DP - D)))
    out = _emb_kernel(xp, tfmt, pp)
    return out[:, :, :D]


# speedup vs baseline: 1.1462x; 1.0098x over previous
import functools

import jax
import jax.numpy as jnp
from jax import lax
from jax.experimental import pallas as pl
from jax.experimental.pallas import tpu as pltpu
from jax.experimental.pallas import tpu_sc as plsc

BATCH = 4096
SEQ = 200
SEQ_PAD = 256
D = 64
DP = 128
VOCAB = 1000000
NC = 2
NS = 16
NW = NC * NS
CHUNKS = BATCH // NW
NBUF = 4
STEPS = CHUNKS // NBUF
GATHER_SLICES = ((0, 128), (128, 72))
ROW_UNROLL = 8
TBLK = 32768
FMT_GRID = -(-VOCAB // TBLK)


def _fmt_body(tt_ref, out_ref):
    out_ref[:, 0:D] = tt_ref[...].T


_fmt_kernel = pl.pallas_call(
    _fmt_body,
    grid=(FMT_GRID,),
    in_specs=[pl.BlockSpec((D, TBLK), lambda j: (0, j))],
    out_specs=pl.BlockSpec((TBLK, DP), lambda j: (j, 0)),
    out_shape=jax.ShapeDtypeStruct((VOCAB, DP), jnp.float32),
)


def _emb_body(x_hbm, tok_hbm, pos_hbm, out_hbm, idx_v, buf, pos_v, sems):
    cid = lax.axis_index("c")
    sid = lax.axis_index("s")
    wid = sid * NC + cid
    base_b = wid * CHUNKS
    sem_i, sem_g, sem_o = sems

    pltpu.sync_copy(pos_hbm, pos_v)

    def issue_idx(c, k):
        pltpu.async_copy(x_hbm.at[base_b + c], idx_v.at[k], sem_i.at[k])

    def wait_idx(k):
        pltpu.make_async_copy(x_hbm.at[0], idx_v.at[k], sem_i.at[k]).wait()

    def issue_gathers(k):
        for (o, n) in GATHER_SLICES:
            pltpu.async_copy(
                tok_hbm.at[idx_v.at[k, pl.ds(o, n)]],
                buf.at[k, pl.ds(o, n)],
                sem_g.at[k],
            )

    def wait_gathers(k):
        pltpu.make_async_copy(tok_hbm.at[pl.ds(0, SEQ)], buf.at[k], sem_g.at[k]).wait()

    def issue_out(c, k):
        pltpu.async_copy(buf.at[k], out_hbm.at[base_b + c], sem_o.at[k])

    def wait_out(k):
        pltpu.make_async_copy(buf.at[k], out_hbm.at[0], sem_o.at[k]).wait()

    def add_pos(k):
        def body(i, carry):
            r0 = i * ROW_UNROLL
            for rr in range(ROW_UNROLL):
                for cc in range(D // 16):
                    plsc.addupdate(
                        buf.at[k, r0 + rr, pl.ds(cc * 16, 16)],
                        pos_v[r0 + rr, pl.ds(cc * 16, 16)],
                    )
            return carry
        lax.fori_loop(0, SEQ // ROW_UNROLL, body, 0)

    for k in range(NBUF):
        issue_idx(k, k)
    for k in range(NBUF):
        wait_idx(k)
        issue_gathers(k)
    for k in range(NBUF):
        wait_gathers(k)
        issue_idx(k + NBUF, k)
        add_pos(k)
        issue_out(k, k)

    def turn(t, carry):
        for k in range(NBUF):
            wait_idx(k)
            wait_out(k)
            issue_gathers(k)
        for k in range(NBUF):
            c = t * NBUF + k
            wait_gathers(k)
            issue_idx(jnp.minimum(c + NBUF, CHUNKS - 1), k)
            add_pos(k)
            issue_out(c, k)
        return carry

    lax.fori_loop(1, STEPS, turn, 0)

    for k in range(NBUF):
        wait_out(k)
        wait_idx(k)


@functools.partial(
    pl.kernel,
    out_type=jax.ShapeDtypeStruct((BATCH, SEQ, DP), jnp.float32),
    mesh=plsc.VectorSubcoreMesh(core_axis_name="c", subcore_axis_name="s"),
    compiler_params=pltpu.CompilerParams(use_tc_tiling_on_sc=True),
    scratch_types=[
        pltpu.VMEM((NBUF, SEQ_PAD), jnp.int32),
        pltpu.VMEM((NBUF, SEQ, DP), jnp.float32),
        pltpu.VMEM((SEQ, DP), jnp.float32),
        (
            pltpu.SemaphoreType.DMA((NBUF,)),
            pltpu.SemaphoreType.DMA((NBUF,)),
            pltpu.SemaphoreType.DMA((NBUF,)),
        ),
    ],
)
def _emb_kernel(x_hbm, tok_hbm, pos_hbm, out_hbm, idx_v, buf, pos_v, sems):
    _emb_body(x_hbm, tok_hbm, pos_hbm, out_hbm, idx_v, buf, pos_v, sems)


def kernel(x, token_table, pos_table):
    tfmt = _fmt_kernel(token_table.T)
    xp = jnp.pad(x, ((0, 0), (0, SEQ_PAD - SEQ)))
    pp = jnp.pad(pos_table, ((0, 0), (0, DP - D)))
    out = _emb_kernel(xp, tfmt, pp)
    return out[:, :, :D]
